# baseline (speedup 1.0000x reference)
import functools

import jax
import jax.numpy as jnp
from jax import lax
from jax.experimental import pallas as pl
from jax.experimental.pallas import tpu as pltpu
from jax.experimental.pallas import tpu_sc as plsc

VD = 128
DA = 32
D = 96
NM = 6
NC = 2
NS = 16
NW = NC * NS
L = 16
IW = 128


def _build_fused_table(xt_t, yt_t, zt_t, W, b2d):
    def body(xt, yt, zt, w, bb, out_ref):
        tabs = (xt, yt, zt)
        for m in range(NM):
            a, p = m // 2, m % 2
            j = p * 3 + a
            blk = lax.dot_general(
                tabs[a][:],
                w[DA * j:DA * (j + 1), :],
                (((0,), (0,)), ((), ())),
                preferred_element_type=jnp.float32,
            )
            if m == 0:
                blk = blk + bb[:]
            out_ref[VD * m:VD * (m + 1), :] = blk.astype(jnp.bfloat16)

    return pl.pallas_call(
        body,
        out_shape=jax.ShapeDtypeStruct((NM * VD, D), jnp.bfloat16),
    )(xt_t, yt_t, zt_t, W, b2d)


def _sc_gather_sum(pos4d, t, batch):
    nblk = batch // IW
    bpw = nblk // NW
    mesh = plsc.VectorSubcoreMesh(
        core_axis_name="c", subcore_axis_name="s", num_cores=NC, num_subcores=NS
    )

    @functools.partial(
        pl.kernel,
        out_type=jax.ShapeDtypeStruct((batch, IW), jnp.float32),
        mesh=mesh,
        scratch_types=[
            pltpu.VMEM((3, bpw, 2, IW), jnp.int32),
            pltpu.VMEM((4, 3 * IW, D), jnp.bfloat16),
            pltpu.VMEM((2, IW, D), jnp.float32),
            pltpu.SemaphoreType.DMA,
            pltpu.SemaphoreType.DMA,
            pltpu.SemaphoreType.DMA,
            pltpu.SemaphoreType.DMA,
            pltpu.SemaphoreType.DMA,
            pltpu.SemaphoreType.DMA,
        ],
        compiler_params=pltpu.CompilerParams(
            needs_layout_passes=False, use_tc_tiling_on_sc=False
        ),
    )
    def k(pos_hbm, t_hbm, out_hbm, idx_v, gath_v, out_v, g0, g1, g2, g3, o0, o1, *_):
        wid = lax.axis_index("s") * NC + lax.axis_index("c")
        blk0 = wid * bpw
        gsem = (g0, g1, g2, g3)
        osem = (o0, o1)
        pltpu.sync_copy(pos_hbm.at[:, pl.ds(blk0, bpw)], idx_v)
        for a in range(3):
            for p in range(2):
                off = (a * 2 + p) * VD
                for ib in range(bpw):
                    for s in range(IW // L):
                        sl = idx_v[a, ib, p, pl.ds(s * L, L)]
                        idx_v[a, ib, p, pl.ds(s * L, L)] = sl + off

        def fire(g, buf):
            ib, p = g // 2, g % 2
            return [
                pltpu.async_copy(
                    t_hbm.at[idx_v.at[a, ib, p]],
                    gath_v.at[buf, pl.ds(a * IW, IW), :],
                    gsem[buf],
                )
                for a in range(3)
            ]

        ngroups = 2 * bpw
        depth = 3
        odesc = [None, None]
        pend = {g: fire(g, g % 4) for g in range(depth)}
        for g in range(ngroups):
            ib, p = g // 2, g % 2
            buf = g % 4
            par = ib % 2
            if g + depth < ngroups:
                pend[g + depth] = fire(g + depth, (g + depth) % 4)
            for cp in pend.pop(g):
                cp.wait()
            if p == 0 and odesc[par] is not None:
                odesc[par].wait()
                odesc[par] = None

            @pl.loop(0, IW, unroll=4)
            def _(e, p=p, buf=buf, par=par):
                for g3 in range(3):
                    acc_a = acc_b = None
                    for a in range(3):
                        v = gath_v[buf, a * IW + e, pl.ds(g3 * 32, 32)]
                        pa, pb = plsc.unpack(
                            v, format=plsc.PackFormat.INTERLEAVED
                        )
                        acc_a = pa if acc_a is None else acc_a + pa
                        acc_b = pb if acc_b is None else acc_b + pb
                    lo = pl.ds(g3 * 32, L)
                    hi = pl.ds(g3 * 32 + L, L)
                    if p == 0:
                        out_v[par, e, lo] = acc_a
                        out_v[par, e, hi] = acc_b
                    else:
                        out_v[par, e, lo] = jnp.maximum(
                            out_v[par, e, lo] + acc_a, 0.0
                        )
                        out_v[par, e, hi] = jnp.maximum(
                            out_v[par, e, hi] + acc_b, 0.0
                        )

            if p == 1:
                odesc[par] = pltpu.async_copy(
                    out_v.at[par],
                    out_hbm.at[pl.ds((blk0 + ib) * IW, IW), pl.ds(0, D)],
                    osem[par],
                )
        for dsc in odesc:
            if dsc is not None:
                dsc.wait()

    return k(pos4d, t)


def kernel(positions, x_table, y_table, z_table, W, b):
    batch = positions.shape[0]
    perm = []
    for g3 in range(3):
        for r in range(L):
            perm.extend((g3 * 32 + r, g3 * 32 + L + r))
    perm = jnp.array(perm, dtype=jnp.int32)
    t = _build_fused_table(
        x_table.T, y_table.T, z_table.T, W[:, perm], b[perm].reshape(1, D)
    )
    pos4d = (
        positions.transpose(2, 0, 1)
        .reshape(3, batch // IW, IW, 2)
        .transpose(0, 1, 3, 2)
    )
    out128 = _sc_gather_sum(pos4d, t, batch)
    return out128[:, :D]


# speedup vs baseline: 1.3391x; 1.3391x over previous
import functools

import jax
import jax.numpy as jnp
from jax import lax
from jax.experimental import pallas as pl
from jax.experimental.pallas import tpu as pltpu
from jax.experimental.pallas import tpu_sc as plsc

VD = 128
DA = 32
D = 96
PW = 128
NC = 2
NS = 16
NW = NC * NS
L = 16
IW = 128
HW = 64
VV = VD * VD


def _build_pair_tables(xt_t, yt_t, zt_t, W, b2d):
    def body(xt, yt, zt, w, bb, out_ref):
        tabs = (xt, yt, zt)
        for a3 in range(3):
            blk_p0 = lax.dot_general(
                tabs[a3][:], w[DA * a3:DA * (a3 + 1), :],
                (((0,), (0,)), ((), ())),
                preferred_element_type=jnp.float32,
            )
            blk_p1 = lax.dot_general(
                tabs[a3][:], w[DA * (3 + a3):DA * (4 + a3), :],
                (((0,), (0,)), ((), ())),
                preferred_element_type=jnp.float32,
            )
            if a3 == 0:
                blk_p0 = blk_p0 + bb[:]
            for v in range(VD):
                row0 = a3 * VV + v * VD
                out_ref[row0:row0 + VD, 0:D] = blk_p0[v:v + 1, :] + blk_p1

    return pl.pallas_call(
        body,
        out_shape=jax.ShapeDtypeStruct((3 * VV, PW), jnp.float32),
    )(xt_t, yt_t, zt_t, W, b2d)


def _sc_gather_sum(pos4d, t2, batch):
    nblk = batch // IW
    bpw = nblk // NW
    mesh = plsc.VectorSubcoreMesh(
        core_axis_name="c", subcore_axis_name="s", num_cores=NC, num_subcores=NS
    )

    @functools.partial(
        pl.kernel,
        out_type=jax.ShapeDtypeStruct((batch, PW), jnp.float32),
        mesh=mesh,
        scratch_types=[
            pltpu.VMEM((3, bpw, 2, IW), jnp.int32),
            pltpu.VMEM((3, bpw, 2, HW), jnp.int32),
            pltpu.VMEM((2, 3 * HW, PW), jnp.float32),
            pltpu.VMEM((2, IW, PW), jnp.float32),
            pltpu.SemaphoreType.DMA,
            pltpu.SemaphoreType.DMA,
            pltpu.SemaphoreType.DMA,
            pltpu.SemaphoreType.DMA,
        ],
        compiler_params=pltpu.CompilerParams(
            needs_layout_passes=False, use_tc_tiling_on_sc=False
        ),
    )
    def k(pos_hbm, t2_hbm, out_hbm, idx_v, idxc_v, gath_v, out_v,
          g0, g1, o0, o1, *_):
        wid = lax.axis_index("s") * NC + lax.axis_index("c")
        blk0 = wid * bpw
        gsem = (g0, g1)
        osem = (o0, o1)
        pltpu.sync_copy(pos_hbm.at[:, pl.ds(blk0, bpw)], idx_v)
        for a in range(3):
            for ib in range(bpw):
                for h in range(2):
                    for s in range(HW // L):
                        src = pl.ds(h * HW + s * L, L)
                        idxc_v[a, ib, h, pl.ds(s * L, L)] = (
                            idx_v[a, ib, 0, src] * VD
                            + idx_v[a, ib, 1, src]
                            + a * VV
                        )

        def fire(hr, buf):
            ib, h = hr // 2, hr % 2
            return [
                pltpu.async_copy(
                    t2_hbm.at[idxc_v.at[a, ib, h]],
                    gath_v.at[buf, pl.ds(a * HW, HW), :],
                    gsem[buf],
                )
                for a in range(3)
            ]

        nrounds = 2 * bpw
        odesc = [None, None]
        pend = fire(0, 0)
        for hr in range(nrounds):
            ib, h = hr // 2, hr % 2
            buf = hr % 2
            par = ib % 2
            nxt = fire(hr + 1, 1 - buf) if hr + 1 < nrounds else []
            for cp in pend:
                cp.wait()
            pend = nxt
            if h == 0 and odesc[par] is not None:
                odesc[par].wait()
                odesc[par] = None

            @pl.loop(0, HW, unroll=4)
            def _(e, buf=buf, par=par, h=h):
                for cb in range(D // L):
                    sl = pl.ds(cb * L, L)
                    acc = gath_v[buf, e, sl]
                    for a in range(1, 3):
                        acc = acc + gath_v[buf, a * HW + e, sl]
                    out_v[par, h * HW + e, sl] = jnp.maximum(acc, 0.0)

            if h == 1:
                odesc[par] = pltpu.async_copy(
                    out_v.at[par],
                    out_hbm.at[pl.ds((blk0 + ib) * IW, IW), :],
                    osem[par],
                )
        for dsc in odesc:
            if dsc is not None:
                dsc.wait()

    return k(pos4d, t2)


def kernel(positions, x_table, y_table, z_table, W, b):
    batch = positions.shape[0]
    t2 = _build_pair_tables(
        x_table.T, y_table.T, z_table.T, W, b.reshape(1, D)
    )
    pos4d = (
        positions.transpose(2, 0, 1)
        .reshape(3, batch // IW, IW, 2)
        .transpose(0, 1, 3, 2)
    )
    out128 = _sc_gather_sum(pos4d, t2, batch)
    return out128[:, :D]


# speedup vs baseline: 1.3717x; 1.0244x over previous
import functools

import jax
import jax.numpy as jnp
from jax import lax
from jax.experimental import pallas as pl
from jax.experimental.pallas import tpu as pltpu
from jax.experimental.pallas import tpu_sc as plsc

VD = 128
DA = 32
D = 96
PW = 128
NC = 2
NS = 16
NW = NC * NS
L = 16
IW = 128
HW = 64
VV = VD * VD


def _build_pair_tables(xt_t, yt_t, zt_t, W, b2d):
    def body(xt, yt, zt, w, bb, out_ref):
        a3 = pl.program_id(0)
        flag = (a3 == 0).astype(jnp.float32)
        tab = (
            xt[:] * (a3 == 0).astype(jnp.float32)
            + yt[:] * (a3 == 1).astype(jnp.float32)
            + zt[:] * (a3 == 2).astype(jnp.float32)
        )
        blk_p0 = lax.dot_general(
            tab, w[pl.ds(a3 * DA, DA), :],
            (((0,), (0,)), ((), ())),
            preferred_element_type=jnp.float32,
        ) + bb[:] * flag
        blk_p1 = lax.dot_general(
            tab, w[pl.ds((a3 + 3) * DA, DA), :],
            (((0,), (0,)), ((), ())),
            preferred_element_type=jnp.float32,
        )
        for v in range(VD):
            out_ref[v * VD:(v + 1) * VD, 0:D] = blk_p0[v:v + 1, :] + blk_p1

    return pl.pallas_call(
        body,
        grid=(3,),
        in_specs=[
            pl.BlockSpec((DA, VD), lambda a: (0, 0)),
            pl.BlockSpec((DA, VD), lambda a: (0, 0)),
            pl.BlockSpec((DA, VD), lambda a: (0, 0)),
            pl.BlockSpec((2 * D, D), lambda a: (0, 0)),
            pl.BlockSpec((1, D), lambda a: (0, 0)),
        ],
        out_specs=pl.BlockSpec((VV, PW), lambda a: (a, 0)),
        out_shape=jax.ShapeDtypeStruct((3 * VV, PW), jnp.float32),
    )(xt_t, yt_t, zt_t, W, b2d)


def _sc_gather_sum(pos4d, t2, batch):
    nblk = batch // IW
    bpw = nblk // NW
    mesh = plsc.VectorSubcoreMesh(
        core_axis_name="c", subcore_axis_name="s", num_cores=NC, num_subcores=NS
    )

    @functools.partial(
        pl.kernel,
        out_type=jax.ShapeDtypeStruct((batch, PW), jnp.float32),
        mesh=mesh,
        scratch_types=[
            pltpu.VMEM((3, bpw, 2, IW), jnp.int32),
            pltpu.VMEM((3, bpw, 2, HW), jnp.int32),
            pltpu.VMEM((2, 3 * HW, PW), jnp.float32),
            pltpu.VMEM((2, IW, PW), jnp.float32),
            pltpu.SemaphoreType.DMA,
            pltpu.SemaphoreType.DMA,
            pltpu.SemaphoreType.DMA,
            pltpu.SemaphoreType.DMA,
        ],
        compiler_params=pltpu.CompilerParams(
            needs_layout_passes=False, use_tc_tiling_on_sc=False
        ),
    )
    def k(pos_hbm, t2_hbm, out_hbm, idx_v, idxc_v, gath_v, out_v,
          g0, g1, o0, o1, *_):
        wid = lax.axis_index("s") * NC + lax.axis_index("c")
        blk0 = wid * bpw
        gsem = (g0, g1)
        osem = (o0, o1)
        pltpu.sync_copy(pos_hbm.at[:, pl.ds(blk0, bpw)], idx_v)
        for a in range(3):
            for ib in range(bpw):
                for h in range(2):
                    for s in range(HW // L):
                        src = pl.ds(h * HW + s * L, L)
                        idxc_v[a, ib, h, pl.ds(s * L, L)] = (
                            idx_v[a, ib, 0, src] * VD
                            + idx_v[a, ib, 1, src]
                            + a * VV
                        )

        def fire(hr, buf):
            ib, h = hr // 2, hr % 2
            return [
                pltpu.async_copy(
                    t2_hbm.at[idxc_v.at[a, ib, h]],
                    gath_v.at[buf, pl.ds(a * HW, HW), :],
                    gsem[buf],
                )
                for a in range(3)
            ]

        nrounds = 2 * bpw
        odesc = [None, None]
        pend = fire(0, 0)
        for hr in range(nrounds):
            ib, h = hr // 2, hr % 2
            buf = hr % 2
            par = ib % 2
            nxt = fire(hr + 1, 1 - buf) if hr + 1 < nrounds else []
            for cp in pend:
                cp.wait()
            pend = nxt
            if h == 0 and odesc[par] is not None:
                odesc[par].wait()
                odesc[par] = None

            @pl.loop(0, HW, unroll=4)
            def _(e, buf=buf, par=par, h=h):
                for cb in range(D // L):
                    sl = pl.ds(cb * L, L)
                    acc = gath_v[buf, e, sl]
                    for a in range(1, 3):
                        acc = acc + gath_v[buf, a * HW + e, sl]
                    out_v[par, h * HW + e, sl] = jnp.maximum(acc, 0.0)

            if h == 1:
                odesc[par] = pltpu.async_copy(
                    out_v.at[par],
                    out_hbm.at[pl.ds((blk0 + ib) * IW, IW), :],
                    osem[par],
                )
        for dsc in odesc:
            if dsc is not None:
                dsc.wait()

    return k(pos4d, t2)


def kernel(positions, x_table, y_table, z_table, W, b):
    batch = positions.shape[0]
    t2 = _build_pair_tables(
        x_table.T, y_table.T, z_table.T, W, b.reshape(1, D)
    )
    pos4d = (
        positions.transpose(2, 0, 1)
        .reshape(3, batch // IW, IW, 2)
        .transpose(0, 1, 3, 2)
    )
    out128 = _sc_gather_sum(pos4d, t2, batch)
    return out128[:, :D]


# speedup vs baseline: 1.3727x; 1.0008x over previous
import functools

import jax
import jax.numpy as jnp
from jax import lax
from jax.experimental import pallas as pl
from jax.experimental.pallas import tpu as pltpu
from jax.experimental.pallas import tpu_sc as plsc

VD = 128
DA = 32
D = 96
PW = 128
NC = 2
NS = 16
NW = NC * NS
L = 16
IW = 128
HW = 64
VV = VD * VD


def _build_pair_tables(xt_t, yt_t, zt_t, W, b2d):
    def body(xt, yt, zt, w, bb, out_ref):
        a3 = pl.program_id(0)
        flag = (a3 == 0).astype(jnp.float32)
        sel = [(a3 == i).astype(jnp.float32) for i in range(3)]
        tab = xt[:] * sel[0] + yt[:] * sel[1] + zt[:] * sel[2]
        blk_p0 = lax.dot_general(
            tab, w[pl.ds(a3 * DA, DA), :],
            (((0,), (0,)), ((), ())),
            preferred_element_type=jnp.float32,
        ) + bb[:] * flag
        blk_p1 = lax.dot_general(
            tab, w[pl.ds((a3 + 3) * DA, DA), :],
            (((0,), (0,)), ((), ())),
            preferred_element_type=jnp.float32,
        )
        for v in range(VD):
            out_ref[v * VD:(v + 1) * VD, 0:D] = blk_p0[v:v + 1, :] + blk_p1

    return pl.pallas_call(
        body,
        grid=(3,),
        in_specs=[
            pl.BlockSpec((DA, VD), lambda a: (0, 0)),
            pl.BlockSpec((DA, VD), lambda a: (0, 0)),
            pl.BlockSpec((DA, VD), lambda a: (0, 0)),
            pl.BlockSpec((2 * D, D), lambda a: (0, 0)),
            pl.BlockSpec((1, D), lambda a: (0, 0)),
        ],
        out_specs=pl.BlockSpec((VV, PW), lambda a: (a, 0)),
        out_shape=jax.ShapeDtypeStruct((3 * VV, PW), jnp.float32),
    )(xt_t, yt_t, zt_t, W, b2d)


def _sc_gather_sum(pos4d, t2, batch):
    nblk = batch // IW
    bpw = nblk // NW
    mesh = plsc.VectorSubcoreMesh(
        core_axis_name="c", subcore_axis_name="s", num_cores=NC, num_subcores=NS
    )

    @functools.partial(
        pl.kernel,
        out_type=jax.ShapeDtypeStruct((batch, PW), jnp.float32),
        mesh=mesh,
        scratch_types=[
            pltpu.VMEM((3, bpw, 2, IW), jnp.int32),
            pltpu.VMEM((3, bpw, 2, HW), jnp.int32),
            pltpu.VMEM((2, 3 * HW, PW), jnp.float32),
            pltpu.VMEM((2, IW, PW), jnp.float32),
            pltpu.SemaphoreType.DMA,
            pltpu.SemaphoreType.DMA,
            pltpu.SemaphoreType.DMA,
            pltpu.SemaphoreType.DMA,
        ],
        compiler_params=pltpu.CompilerParams(
            needs_layout_passes=False, use_tc_tiling_on_sc=False
        ),
    )
    def k(pos_hbm, t2_hbm, out_hbm, idx_v, idxc_v, gath_v, out_v,
          g0, g1, o0, o1, *_):
        wid = lax.axis_index("s") * NC + lax.axis_index("c")
        blk0 = wid * bpw
        gsem = (g0, g1)
        osem = (o0, o1)
        pltpu.sync_copy(pos_hbm.at[:, pl.ds(blk0, bpw)], idx_v)
        for a in range(3):
            for ib in range(bpw):
                for h in range(2):
                    for s in range(HW // L):
                        src = pl.ds(h * HW + s * L, L)
                        idxc_v[a, ib, h, pl.ds(s * L, L)] = (
                            idx_v[a, ib, 0, src] * VD
                            + idx_v[a, ib, 1, src]
                            + a * VV
                        )

        def fire(hr, buf):
            ib, h = hr // 2, hr % 2
            return [
                pltpu.async_copy(
                    t2_hbm.at[idxc_v.at[a, ib, h]],
                    gath_v.at[buf, pl.ds(a * HW, HW), :],
                    gsem[buf],
                )
                for a in range(3)
            ]

        nrounds = 2 * bpw
        odesc = [None, None]
        pend = fire(0, 0)
        for hr in range(nrounds):
            ib, h = hr // 2, hr % 2
            buf = hr % 2
            par = ib % 2
            nxt = fire(hr + 1, 1 - buf) if hr + 1 < nrounds else []
            for cp in pend:
                cp.wait()
            pend = nxt
            if h == 0 and odesc[par] is not None:
                odesc[par].wait()
                odesc[par] = None

            @pl.loop(0, HW, unroll=4)
            def _(e, buf=buf, par=par, h=h):
                for cb in range(D // L):
                    sl = pl.ds(cb * L, L)
                    acc = gath_v[buf, e, sl]
                    for a in range(1, 3):
                        acc = acc + gath_v[buf, a * HW + e, sl]
                    out_v[par, h * HW + e, sl] = jnp.maximum(acc, 0.0)

            if h == 1:
                odesc[par] = pltpu.async_copy(
                    out_v.at[par],
                    out_hbm.at[pl.ds((blk0 + ib) * IW, IW), :],
                    osem[par],
                )
        for dsc in odesc:
            if dsc is not None:
                dsc.wait()

    return k(pos4d, t2)


def kernel(positions, x_table, y_table, z_table, W, b):
    batch = positions.shape[0]
    t2 = _build_pair_tables(
        x_table.T, y_table.T, z_table.T, W, b.reshape(1, D)
    )
    pos4d = (
        positions.transpose(2, 0, 1)
        .reshape(3, batch // IW, IW, 2)
        .transpose(0, 1, 3, 2)
    )
    out128 = _sc_gather_sum(pos4d, t2, batch)
    return out128[:, :D]
